# baseline (speedup 1.0000x reference)
import functools

import jax
import jax.numpy as jnp
from jax import lax
from jax.experimental import pallas as pl
from jax.experimental.pallas import tpu as pltpu
from jax.experimental.pallas import tpu_sc as plsc

N = 10000
E = 320000
D = 128

NC = 2
NS = 16
K = 128
CHT = 160
WIN = 32
NWD = CHT // WIN
CPAD = NS * CHT
EP = CPAD * K
NP = 10240
RPS = NP // NS
BR = 1000
GRID = N // BR


def _sc_agg_body(with_deg, y_hbm, src_hbm, dst_hbm, zeros_hbm, ones_hbm,
                 z_out, z_sh, src_w, dst_w, rows0, rows1, sem0, sem1):
    c = lax.axis_index("c")
    s = lax.axis_index("s")
    row0 = s * RPS
    base = s * CHT

    rowsb = (rows0, rows1)
    semb = (sem0, sem1)

    @pl.when(c == 0)
    def _agg_loop():
        pltpu.sync_copy(zeros_hbm, z_sh.at[pl.ds(row0, RPS)])
        plsc.subcore_barrier()

        def window(w, _):
            base_w = base + w * WIN
            pltpu.sync_copy(src_hbm.at[pl.ds(base_w, WIN)], src_w)
            pltpu.sync_copy(dst_hbm.at[pl.ds(base_w, WIN)], dst_w)
            for b in range(2):
                pltpu.async_copy(y_hbm.at[src_w.at[b]], rowsb[b], semb[b])

            def steps(t, _):
                for b in range(2):
                    j = t * 2 + b
                    pltpu.make_async_copy(y_hbm.at[src_w.at[j]],
                                          rowsb[b], semb[b]).wait()
                    pltpu.sync_copy(rowsb[b], z_sh.at[dst_w.at[j]], add=True)
                    nxt = j + 2

                    @pl.when(nxt < WIN)
                    def _():
                        pltpu.async_copy(y_hbm.at[src_w.at[nxt]],
                                         rowsb[b], semb[b])
                return 0

            lax.fori_loop(0, WIN // 2, steps, 0)
            return 0

        lax.fori_loop(0, NWD, window, 0)
        plsc.subcore_barrier()
        pltpu.sync_copy(z_sh.at[pl.ds(row0, RPS)],
                        z_out.at[0, pl.ds(row0, RPS)])

    if with_deg:
        @pl.when(c == 1)
        def _deg_loop():
            pltpu.sync_copy(zeros_hbm, z_sh.at[pl.ds(row0, RPS)])
            plsc.subcore_barrier()
            pltpu.sync_copy(ones_hbm, rows0)

            def window(w, _):
                base_w = base + w * WIN
                pltpu.sync_copy(dst_hbm.at[pl.ds(base_w, WIN)], dst_w)

                def steps(j, _):
                    pltpu.sync_copy(rows0, z_sh.at[dst_w.at[j]], add=True)
                    return 0

                lax.fori_loop(0, WIN, steps, 0)
                return 0

            lax.fori_loop(0, NWD, window, 0)
            plsc.subcore_barrier()
            pltpu.sync_copy(z_sh.at[pl.ds(row0, RPS)],
                            z_out.at[1, pl.ds(row0, RPS)])


@functools.lru_cache(maxsize=None)
def _make_sc_agg(with_deg):
    mesh = plsc.VectorSubcoreMesh(core_axis_name="c", subcore_axis_name="s")
    return pl.kernel(
        functools.partial(_sc_agg_body, with_deg),
        mesh=mesh,
        out_type=[jax.ShapeDtypeStruct((NC, NP, D), jnp.float32)],
        scratch_types=[
            pltpu.VMEM_SHARED((NP, D), jnp.float32),
            pltpu.VMEM((WIN, K), jnp.int32),
            pltpu.VMEM((WIN, K), jnp.int32),
            pltpu.VMEM((K, D), jnp.float32),
            pltpu.VMEM((K, D), jnp.float32),
            pltpu.SemaphoreType.DMA,
            pltpu.SemaphoreType.DMA,
        ],
    )


def _tc_pre_body(x_ref, ws_ref, wn_ref, b_ref, s_out, y_out):
    x = x_ref[...]
    s_out[...] = (jnp.dot(x, ws_ref[...], preferred_element_type=jnp.float32)
                  + b_ref[0:1, :])
    y_out[...] = jnp.dot(x, wn_ref[...], preferred_element_type=jnp.float32)


def _tc_mid_body(s1_ref, z_ref, deg_ref, ws_ref, wn_ref, b_ref, s2_out, y2_out):
    inv = 1.0 / jnp.maximum(deg_ref[0, :, 0:1], 1.0)
    h = jnp.maximum(s1_ref[...] + z_ref[0] * inv, 0.0)
    s2_out[...] = (jnp.dot(h, ws_ref[...], preferred_element_type=jnp.float32)
                   + b_ref[0:1, :])
    y2_out[...] = jnp.dot(h, wn_ref[...], preferred_element_type=jnp.float32)


def _tc_post_body(s2_ref, z_ref, deg_ref, out_ref):
    inv = 1.0 / jnp.maximum(deg_ref[0, :, 0:1], 1.0)
    out_ref[...] = s2_ref[...] + z_ref[0] * inv


_row_spec = pl.BlockSpec((BR, D), lambda i: (i, 0))
_w_spec = pl.BlockSpec((D, D), lambda i: (0, 0))
_b_spec = pl.BlockSpec((8, D), lambda i: (0, 0))
_z0_spec = pl.BlockSpec((1, BR, D), lambda i: (0, i, 0))
_z1_spec = pl.BlockSpec((1, BR, D), lambda i: (1, i, 0))

_tc_pre = pl.pallas_call(
    _tc_pre_body,
    grid=(GRID,),
    in_specs=[_row_spec, _w_spec, _w_spec, _b_spec],
    out_specs=[_row_spec, _row_spec],
    out_shape=[jax.ShapeDtypeStruct((N, D), jnp.float32)] * 2,
)

_tc_mid = pl.pallas_call(
    _tc_mid_body,
    grid=(GRID,),
    in_specs=[_row_spec, _z0_spec, _z1_spec, _w_spec, _w_spec, _b_spec],
    out_specs=[_row_spec, _row_spec],
    out_shape=[jax.ShapeDtypeStruct((N, D), jnp.float32)] * 2,
)

_tc_post = pl.pallas_call(
    _tc_post_body,
    grid=(GRID,),
    in_specs=[_row_spec, _z0_spec, _z1_spec],
    out_specs=_row_spec,
    out_shape=jax.ShapeDtypeStruct((N, D), jnp.float32),
)


def kernel(features, edge_index, W1_self, W1_neigh, b1, W2_self, W2_neigh, b2):
    pad = EP - E
    src_r = jnp.concatenate(
        [edge_index[0], jnp.zeros((pad,), jnp.int32)]).reshape(CPAD, K)
    dst_r = jnp.concatenate(
        [edge_index[1], jnp.full((pad,), N, jnp.int32)]).reshape(CPAD, K)
    zeros_c = jnp.zeros((RPS, D), jnp.float32)
    ones_c = jnp.ones((K, D), jnp.float32)
    b1r = jnp.broadcast_to(b1.reshape(1, D), (8, D))
    b2r = jnp.broadcast_to(b2.reshape(1, D), (8, D))

    s1, y1 = _tc_pre(features, W1_self, W1_neigh, b1r)
    (zd1,) = _make_sc_agg(True)(y1, src_r, dst_r, zeros_c, ones_c)
    s2, y2 = _tc_mid(s1, zd1, zd1, W2_self, W2_neigh, b2r)
    (z2,) = _make_sc_agg(False)(y2, src_r, dst_r, zeros_c, ones_c)
    return _tc_post(s2, z2, zd1)


# speedup vs baseline: 1.0801x; 1.0801x over previous
import functools

import jax
import jax.numpy as jnp
from jax import lax
from jax.experimental import pallas as pl
from jax.experimental.pallas import tpu as pltpu
from jax.experimental.pallas import tpu_sc as plsc

N = 10000
E = 320000
D = 128

NC = 2
NS = 16
K = 128
CH0 = 128
CH1 = 32
CHD = 80
WIN = 32
CPAD = NS * (CH0 + CH1)
EP = CPAD * K
NP = 10240
RPS = NP // NS
BR = 1000
GRID = N // BR


def _sc_deg_body(dst_hbm, zeros_hbm, ones_hbm, deg_out, deg_sh, dst_v, rows_v):
    c = lax.axis_index("c")
    s = lax.axis_index("s")
    wid = s * NC + c
    row0 = s * RPS

    pltpu.sync_copy(zeros_hbm, deg_sh.at[pl.ds(row0, RPS)])
    plsc.subcore_barrier()

    pltpu.sync_copy(ones_hbm, rows_v)
    pltpu.sync_copy(dst_hbm.at[wid], dst_v)

    def step(i, _):
        pltpu.sync_copy(rows_v, deg_sh.at[dst_v.at[i]], add=True)
        return 0

    lax.fori_loop(0, CHD, step, 0)
    plsc.subcore_barrier()

    pltpu.sync_copy(deg_sh.at[pl.ds(row0, RPS)],
                    deg_out.at[c, pl.ds(row0, RPS)])


@functools.lru_cache(maxsize=None)
def _make_sc_deg():
    mesh = plsc.VectorSubcoreMesh(core_axis_name="c", subcore_axis_name="s")
    return pl.kernel(
        _sc_deg_body,
        mesh=mesh,
        out_type=[jax.ShapeDtypeStruct((NC, NP, D), jnp.float32)],
        scratch_types=[
            pltpu.VMEM_SHARED((NP, D), jnp.float32),
            pltpu.VMEM((CHD, K), jnp.int32),
            pltpu.VMEM((K, D), jnp.float32),
        ],
    )


def _sc_agg_body(y_hbm, src_hbm, dst_hbm, zeros_hbm,
                 z_out, z_sh, src_w, dst_w, rows0, rows1, sem0, sem1):
    c = lax.axis_index("c")
    s = lax.axis_index("s")
    row0 = s * RPS
    nw = jnp.where(c == 0, CH0 // WIN, CH1 // WIN)
    base = jnp.where(c == 0, s * CH0, NS * CH0 + s * CH1)

    pltpu.sync_copy(zeros_hbm, z_sh.at[pl.ds(row0, RPS)])
    plsc.subcore_barrier()

    rowsb = (rows0, rows1)
    semb = (sem0, sem1)

    def window(w, _):
        base_w = base + w * WIN
        pltpu.sync_copy(src_hbm.at[pl.ds(base_w, WIN)], src_w)
        pltpu.sync_copy(dst_hbm.at[pl.ds(base_w, WIN)], dst_w)
        for b in range(2):
            pltpu.async_copy(y_hbm.at[src_w.at[b]], rowsb[b], semb[b])

        def steps(t, _):
            for b in range(2):
                j = t * 2 + b
                pltpu.make_async_copy(y_hbm.at[src_w.at[j]],
                                      rowsb[b], semb[b]).wait()
                pltpu.sync_copy(rowsb[b], z_sh.at[dst_w.at[j]], add=True)
                nxt = j + 2

                @pl.when(nxt < WIN)
                def _():
                    pltpu.async_copy(y_hbm.at[src_w.at[nxt]],
                                     rowsb[b], semb[b])
            return 0

        lax.fori_loop(0, WIN // 2, steps, 0)
        return 0

    lax.fori_loop(0, nw, window, 0)
    plsc.subcore_barrier()

    pltpu.sync_copy(z_sh.at[pl.ds(row0, RPS)],
                    z_out.at[c, pl.ds(row0, RPS)])


@functools.lru_cache(maxsize=None)
def _make_sc_agg():
    mesh = plsc.VectorSubcoreMesh(core_axis_name="c", subcore_axis_name="s")
    return pl.kernel(
        _sc_agg_body,
        mesh=mesh,
        out_type=[jax.ShapeDtypeStruct((NC, NP, D), jnp.float32)],
        scratch_types=[
            pltpu.VMEM_SHARED((NP, D), jnp.float32),
            pltpu.VMEM((WIN, K), jnp.int32),
            pltpu.VMEM((WIN, K), jnp.int32),
            pltpu.VMEM((K, D), jnp.float32),
            pltpu.VMEM((K, D), jnp.float32),
            pltpu.SemaphoreType.DMA,
            pltpu.SemaphoreType.DMA,
        ],
    )


def _tc_pre_body(x_ref, ws_ref, wn_ref, b_ref, s_out, y_out):
    x = x_ref[...]
    s_out[...] = (jnp.dot(x, ws_ref[...], preferred_element_type=jnp.float32)
                  + b_ref[0:1, :])
    y_out[...] = jnp.dot(x, wn_ref[...], preferred_element_type=jnp.float32)


def _tc_mid_body(s1_ref, z_ref, deg_ref, ws_ref, wn_ref, b_ref, s2_out, y2_out):
    deg = deg_ref[0, :, 0:1] + deg_ref[1, :, 0:1]
    inv = 1.0 / jnp.maximum(deg, 1.0)
    h = jnp.maximum(s1_ref[...] + (z_ref[0] + z_ref[1]) * inv, 0.0)
    s2_out[...] = (jnp.dot(h, ws_ref[...], preferred_element_type=jnp.float32)
                   + b_ref[0:1, :])
    y2_out[...] = jnp.dot(h, wn_ref[...], preferred_element_type=jnp.float32)


def _tc_post_body(s2_ref, z_ref, deg_ref, out_ref):
    deg = deg_ref[0, :, 0:1] + deg_ref[1, :, 0:1]
    inv = 1.0 / jnp.maximum(deg, 1.0)
    out_ref[...] = s2_ref[...] + (z_ref[0] + z_ref[1]) * inv


_row_spec = pl.BlockSpec((BR, D), lambda i: (i, 0))
_w_spec = pl.BlockSpec((D, D), lambda i: (0, 0))
_b_spec = pl.BlockSpec((8, D), lambda i: (0, 0))
_z_spec = pl.BlockSpec((NC, BR, D), lambda i: (0, i, 0))

_tc_pre = pl.pallas_call(
    _tc_pre_body,
    grid=(GRID,),
    in_specs=[_row_spec, _w_spec, _w_spec, _b_spec],
    out_specs=[_row_spec, _row_spec],
    out_shape=[jax.ShapeDtypeStruct((N, D), jnp.float32)] * 2,
)

_tc_mid = pl.pallas_call(
    _tc_mid_body,
    grid=(GRID,),
    in_specs=[_row_spec, _z_spec, _z_spec, _w_spec, _w_spec, _b_spec],
    out_specs=[_row_spec, _row_spec],
    out_shape=[jax.ShapeDtypeStruct((N, D), jnp.float32)] * 2,
)

_tc_post = pl.pallas_call(
    _tc_post_body,
    grid=(GRID,),
    in_specs=[_row_spec, _z_spec, _z_spec],
    out_specs=_row_spec,
    out_shape=jax.ShapeDtypeStruct((N, D), jnp.float32),
)


def kernel(features, edge_index, W1_self, W1_neigh, b1, W2_self, W2_neigh, b2):
    pad = EP - E
    src_r = jnp.concatenate(
        [edge_index[0], jnp.zeros((pad,), jnp.int32)]).reshape(CPAD, K)
    dst_r = jnp.concatenate(
        [edge_index[1], jnp.full((pad,), N, jnp.int32)]).reshape(CPAD, K)
    zeros_c = jnp.zeros((RPS, D), jnp.float32)
    ones_c = jnp.ones((K, D), jnp.float32)
    b1r = jnp.broadcast_to(b1.reshape(1, D), (8, D))
    b2r = jnp.broadcast_to(b2.reshape(1, D), (8, D))

    dst_d = dst_r.reshape(NC * NS, CHD, K)
    (deg,) = _make_sc_deg()(dst_d, zeros_c, ones_c)
    s1, y1 = _tc_pre(features, W1_self, W1_neigh, b1r)
    (z1,) = _make_sc_agg()(y1, src_r, dst_r, zeros_c)
    s2, y2 = _tc_mid(s1, z1, deg, W2_self, W2_neigh, b2r)
    (z2,) = _make_sc_agg()(y2, src_r, dst_r, zeros_c)
    return _tc_post(s2, z2, deg)
